# baseline (speedup 1.0000x reference)
import jax
import jax.numpy as jnp
from jax import lax
from jax.experimental import pallas as pl
from jax.experimental.pallas import tpu as pltpu

_B0 = 200
_B1 = 400
_CH = ((0, 2048), (2048, 2048), (4096, 2048), (6144, 2048), (8192, 1808))


def _lrelu(v):
    return jnp.where(v >= 0, v, 0.01 * v)


def _mega_body(adj_hbm, x_ref, w1_ref, w2_ref, w3_ref, b1_ref, b2_ref,
               b3_ref, out_ref, adjb_hbm, rd32, ab16, h_ref, u_ref,
               rsem, wsem):
    n = adj_hbm.shape[0]
    nb0 = n // _B0
    nb1 = n // _B1
    bf = jnp.bfloat16
    f = u_ref.shape[1]

    def adj_rd(k, s):
        return pltpu.make_async_copy(
            adj_hbm.at[pl.ds(k * _B0, _B0)], rd32.at[s], rsem.at[s])

    def ab_wr(k, s):
        return pltpu.make_async_copy(
            ab16.at[s], adjb_hbm.at[pl.ds(k * _B1, _B1)], wsem.at[s])

    def ab_rd(k, s):
        return pltpu.make_async_copy(
            adjb_hbm.at[pl.ds(k * _B1, _B1)], ab16.at[s], rsem.at[s])

    adj_rd(0, 0).start()
    adj_rd(1, 1).start()

    u_ref[...] = jnp.dot(
        x_ref[...].astype(bf), w1_ref[...].astype(bf),
        preferred_element_type=jnp.float32).astype(bf)

    def p0_process(k0, s, s4, part, wr_wait):
        adj_rd(k0, s).wait()
        if wr_wait:
            ab_wr(0, s4).wait()
        acc = jnp.zeros((_B0, f), jnp.float32)
        r0 = part * _B0
        for off, w in _CH:
            chb = rd32[s, :, off:off + w].astype(bf)
            ab16[s4, r0:r0 + _B0, off:off + w] = chb
            acc = acc + jnp.dot(chb, u_ref[off:off + w, :],
                                preferred_element_type=jnp.float32)
        h_ref[pl.ds(k0 * _B0, _B0), :] = _lrelu(acc + b1_ref[...])

    for j, (s4, part, ww) in enumerate(
            ((0, 0, False), (0, 1, False), (1, 0, False), (1, 1, False))):
        p0_process(j, j % 2, s4, part, ww)
        if part == 1:
            ab_wr(s4, s4).start()
        adj_rd(j + 2, j % 2).start()

    def p0_quad(q, c):
        for j, (s4, part, ww) in enumerate(
                ((0, 0, True), (0, 1, False), (1, 0, True), (1, 1, False))):
            k0 = 4 * q + j
            p0_process(k0, j % 2, s4, part, ww)
            if part == 1:
                ab_wr(2 * q + s4, s4).start()
            adj_rd(k0 + 2, j % 2).start()
        return c

    lax.fori_loop(1, nb0 // 4, p0_quad, 0)
    p0_process(nb0 - 2, 0, 0, 0, True)
    p0_process(nb0 - 1, 1, 0, 1, False)
    ab_wr(nb1 - 1, 0).start()
    ab_wr(0, 1).wait()
    ab_wr(0, 0).wait()

    def pbf_process(k, s, dst, b_ref, act):
        ab_rd(k, s).wait()
        acc = jnp.dot(ab16[s, :, :], u_ref[...],
                      preferred_element_type=jnp.float32)
        v = acc + b_ref[...]
        if act:
            v = _lrelu(v)
        dst[pl.ds(k * _B1, _B1), :] = v

    def bf_phase(dst, w_ref, b_ref, act):
        ab_rd(0, 0).start()
        ab_rd(1, 1).start()
        u_ref[...] = jnp.dot(
            h_ref[...].astype(bf), w_ref[...].astype(bf),
            preferred_element_type=jnp.float32).astype(bf)

        pbf_process(0, 0, dst, b_ref, act)
        ab_rd(2, 0).start()
        pbf_process(1, 1, dst, b_ref, act)
        ab_rd(3, 1).start()

        def pair(p, c):
            k = 2 * p
            pbf_process(k, 0, dst, b_ref, act)
            ab_rd(k + 2, 0).start()
            pbf_process(k + 1, 1, dst, b_ref, act)

            @pl.when(k + 3 < nb1)
            def _():
                ab_rd(k + 3, 1).start()
            return c

        lax.fori_loop(1, (nb1 - 1) // 2, pair, 0)
        pbf_process(nb1 - 1, 0, dst, b_ref, act)

    bf_phase(h_ref, w2_ref, b2_ref, True)
    bf_phase(out_ref, w3_ref, b3_ref, False)


def kernel(x, adj, W1, b1, W2, b2, W3, b3):
    n, fin = x.shape
    f = W1.shape[1]
    out, _ = pl.pallas_call(
        _mega_body,
        in_specs=[
            pl.BlockSpec(memory_space=pl.ANY),
            pl.BlockSpec((n, fin), lambda: (0, 0)),
            pl.BlockSpec((fin, f), lambda: (0, 0)),
            pl.BlockSpec((f, f), lambda: (0, 0)),
            pl.BlockSpec((f, f), lambda: (0, 0)),
            pl.BlockSpec((1, f), lambda: (0, 0)),
            pl.BlockSpec((1, f), lambda: (0, 0)),
            pl.BlockSpec((1, f), lambda: (0, 0)),
        ],
        out_specs=[
            pl.BlockSpec((n, f), lambda: (0, 0)),
            pl.BlockSpec(memory_space=pl.ANY),
        ],
        out_shape=[
            jax.ShapeDtypeStruct((n, f), jnp.float32),
            jax.ShapeDtypeStruct((n, n), jnp.bfloat16),
        ],
        scratch_shapes=[
            pltpu.VMEM((2, _B0, n), jnp.float32),
            pltpu.VMEM((2, _B1, n), jnp.bfloat16),
            pltpu.VMEM((n, f), jnp.float32),
            pltpu.VMEM((n, f), jnp.bfloat16),
            pltpu.SemaphoreType.DMA((2,)),
            pltpu.SemaphoreType.DMA((2,)),
        ],
        compiler_params=pltpu.CompilerParams(
            vmem_limit_bytes=100 * 1024 * 1024),
    )(adj, x, W1, W2, W3, b1.reshape(1, -1), b2.reshape(1, -1),
      b3.reshape(1, -1))
    return out


# speedup vs baseline: 1.0462x; 1.0462x over previous
import jax
import jax.numpy as jnp
from jax import lax
from jax.experimental import pallas as pl
from jax.experimental.pallas import tpu as pltpu

_B0 = 200
_B1 = 400
_CH = ((0, 2048), (2048, 2048), (4096, 2048), (6144, 2048), (8192, 1808))


def _lrelu(v):
    return jnp.where(v >= 0, v, 0.01 * v)


def _mega_body(adj_hbm, x_ref, w1_ref, w2_ref, w3_ref, b1_ref, b2_ref,
               b3_ref, out_ref, adjb_hbm, rd0, rd1, ab0, ab1, h_ref, u_ref,
               rs0, rs1, ws0, ws1):
    n = adj_hbm.shape[0]
    nb0 = n // _B0
    nb1 = n // _B1
    bf = jnp.bfloat16
    f = u_ref.shape[1]

    rd = (rd0, rd1)
    ab = (ab0, ab1)
    rs = (rs0, rs1)
    ws = (ws0, ws1)

    def adj_rd(k, s):
        return pltpu.make_async_copy(
            adj_hbm.at[pl.ds(k * _B0, _B0)], rd[s], rs[s])

    def ab_wr(k, s):
        return pltpu.make_async_copy(
            ab[s], adjb_hbm.at[pl.ds(k * _B1, _B1)], ws[s])

    def ab_rd(k, s):
        return pltpu.make_async_copy(
            adjb_hbm.at[pl.ds(k * _B1, _B1)], ab[s], rs[s])

    adj_rd(0, 0).start()
    adj_rd(1, 1).start()

    u_ref[...] = jnp.dot(
        x_ref[...].astype(bf), w1_ref[...].astype(bf),
        preferred_element_type=jnp.float32).astype(bf)

    def p0_process(k0, s, s4, part, wr_wait):
        adj_rd(k0, s).wait()
        if wr_wait:
            ab_wr(0, s4).wait()
        acc = jnp.zeros((_B0, f), jnp.float32)
        r0 = part * _B0
        for off, w in _CH:
            chb = rd[s][:, off:off + w].astype(bf)
            ab[s4][r0:r0 + _B0, off:off + w] = chb
            acc = acc + jnp.dot(chb, u_ref[off:off + w, :],
                                preferred_element_type=jnp.float32)
        h_ref[pl.ds(k0 * _B0, _B0), :] = _lrelu(acc + b1_ref[...])

    for j, (s4, part) in enumerate(((0, 0), (0, 1), (1, 0), (1, 1))):
        p0_process(j, j % 2, s4, part, False)
        if part == 1:
            ab_wr(s4, s4).start()
        adj_rd(j + 2, j % 2).start()

    def p0_quad(q, c):
        for j, (s4, part, ww) in enumerate(
                ((0, 0, True), (0, 1, False), (1, 0, True), (1, 1, False))):
            k0 = 4 * q + j
            p0_process(k0, j % 2, s4, part, ww)
            if part == 1:
                ab_wr(2 * q + s4, s4).start()
            adj_rd(k0 + 2, j % 2).start()
        return c

    lax.fori_loop(1, nb0 // 4, p0_quad, 0)
    p0_process(nb0 - 2, 0, 0, 0, True)
    p0_process(nb0 - 1, 1, 0, 1, False)
    ab_wr(nb1 - 1, 0).start()
    ab_wr(0, 1).wait()
    ab_wr(0, 0).wait()

    def pbf_process(k, s, dst, b_ref, act):
        ab_rd(k, s).wait()
        acc = jnp.dot(ab[s][...], u_ref[...],
                      preferred_element_type=jnp.float32)
        v = acc + b_ref[...]
        if act:
            v = _lrelu(v)
        dst[pl.ds(k * _B1, _B1), :] = v

    def bf_phase(dst, w_ref, b_ref, act):
        ab_rd(0, 0).start()
        ab_rd(1, 1).start()
        u_ref[...] = jnp.dot(
            h_ref[...].astype(bf), w_ref[...].astype(bf),
            preferred_element_type=jnp.float32).astype(bf)

        pbf_process(0, 0, dst, b_ref, act)
        ab_rd(2, 0).start()
        pbf_process(1, 1, dst, b_ref, act)
        ab_rd(3, 1).start()

        def pair(p, c):
            k = 2 * p
            pbf_process(k, 0, dst, b_ref, act)
            ab_rd(k + 2, 0).start()
            pbf_process(k + 1, 1, dst, b_ref, act)

            @pl.when(k + 3 < nb1)
            def _():
                ab_rd(k + 3, 1).start()
            return c

        lax.fori_loop(1, (nb1 - 1) // 2, pair, 0)
        pbf_process(nb1 - 1, 0, dst, b_ref, act)

    bf_phase(h_ref, w2_ref, b2_ref, True)
    bf_phase(out_ref, w3_ref, b3_ref, False)


def kernel(x, adj, W1, b1, W2, b2, W3, b3):
    n, fin = x.shape
    f = W1.shape[1]
    out, _ = pl.pallas_call(
        _mega_body,
        in_specs=[
            pl.BlockSpec(memory_space=pl.ANY),
            pl.BlockSpec((n, fin), lambda: (0, 0)),
            pl.BlockSpec((fin, f), lambda: (0, 0)),
            pl.BlockSpec((f, f), lambda: (0, 0)),
            pl.BlockSpec((f, f), lambda: (0, 0)),
            pl.BlockSpec((1, f), lambda: (0, 0)),
            pl.BlockSpec((1, f), lambda: (0, 0)),
            pl.BlockSpec((1, f), lambda: (0, 0)),
        ],
        out_specs=[
            pl.BlockSpec((n, f), lambda: (0, 0)),
            pl.BlockSpec(memory_space=pl.ANY),
        ],
        out_shape=[
            jax.ShapeDtypeStruct((n, f), jnp.float32),
            jax.ShapeDtypeStruct((n, n), jnp.bfloat16),
        ],
        scratch_shapes=[
            pltpu.VMEM((_B0, n), jnp.float32),
            pltpu.VMEM((_B0, n), jnp.float32),
            pltpu.VMEM((_B1, n), jnp.bfloat16),
            pltpu.VMEM((_B1, n), jnp.bfloat16),
            pltpu.VMEM((n, f), jnp.float32),
            pltpu.VMEM((n, f), jnp.bfloat16),
            pltpu.SemaphoreType.DMA,
            pltpu.SemaphoreType.DMA,
            pltpu.SemaphoreType.DMA,
            pltpu.SemaphoreType.DMA,
        ],
        compiler_params=pltpu.CompilerParams(
            vmem_limit_bytes=100 * 1024 * 1024),
    )(adj, x, W1, W2, W3, b1.reshape(1, -1), b2.reshape(1, -1),
      b3.reshape(1, -1))
    return out


# speedup vs baseline: 1.2668x; 1.2109x over previous
import jax
import jax.numpy as jnp
from jax.experimental import pallas as pl
from jax.experimental.pallas import tpu as pltpu
from functools import partial

_BR1 = 400
_BR2 = 1000


def _l1_kernel(adj_ref, x_ref, w_ref, b_ref, h_ref, adjb_ref, u_ref):
    @pl.when(pl.program_id(0) == 0)
    def _():
        u_ref[...] = jnp.dot(
            x_ref[...].astype(jnp.bfloat16),
            w_ref[...].astype(jnp.bfloat16),
            preferred_element_type=jnp.float32,
        ).astype(jnp.bfloat16)

    a = adj_ref[...].astype(jnp.bfloat16)
    adjb_ref[...] = a
    acc = jnp.dot(a, u_ref[...], preferred_element_type=jnp.float32)
    acc = acc + b_ref[...]
    h_ref[...] = jnp.where(acc >= 0, acc, 0.01 * acc)


def _layer1(adj, x, w, b):
    n = adj.shape[0]
    f = w.shape[1]
    fin = x.shape[1]
    return pl.pallas_call(
        _l1_kernel,
        grid=(n // _BR1,),
        in_specs=[
            pl.BlockSpec((_BR1, n), lambda i: (i, 0)),
            pl.BlockSpec((n, fin), lambda i: (0, 0)),
            pl.BlockSpec((fin, f), lambda i: (0, 0)),
            pl.BlockSpec((1, f), lambda i: (0, 0)),
        ],
        out_specs=[
            pl.BlockSpec((_BR1, f), lambda i: (i, 0)),
            pl.BlockSpec((_BR1, n), lambda i: (i, 0)),
        ],
        out_shape=[
            jax.ShapeDtypeStruct((n, f), jnp.float32),
            jax.ShapeDtypeStruct((n, n), jnp.bfloat16),
        ],
        scratch_shapes=[pltpu.VMEM((n, f), jnp.bfloat16)],
    )(adj, x, w, b)


def _l23_kernel(adjb_ref, h1_ref, w2_ref, w3_ref, b2_ref, b3_ref, o_ref,
                u_ref, h2_ref):
    l = pl.program_id(0)
    i = pl.program_id(1)

    @pl.when((l == 0) & (i == 0))
    def _():
        u_ref[...] = jnp.dot(
            h1_ref[...].astype(jnp.bfloat16),
            w2_ref[...].astype(jnp.bfloat16),
            preferred_element_type=jnp.float32,
        ).astype(jnp.bfloat16)

    @pl.when((l == 1) & (i == 0))
    def _():
        u_ref[...] = jnp.dot(
            h2_ref[...].astype(jnp.bfloat16),
            w3_ref[...].astype(jnp.bfloat16),
            preferred_element_type=jnp.float32,
        ).astype(jnp.bfloat16)

    acc = jnp.dot(adjb_ref[...], u_ref[...], preferred_element_type=jnp.float32)
    v = acc + jnp.where(l == 0, b2_ref[...], b3_ref[...])
    v = jnp.where(l == 0, jnp.where(v >= 0, v, 0.01 * v), v)
    o_ref[...] = v

    @pl.when(l == 0)
    def _():
        h2_ref[pl.ds(i * _BR2, _BR2), :] = v


def _layer23(adjb, h1, w2, w3, b2, b3):
    n = adjb.shape[0]
    f = w2.shape[1]
    fin = h1.shape[1]
    return pl.pallas_call(
        _l23_kernel,
        grid=(2, n // _BR2),
        in_specs=[
            pl.BlockSpec((_BR2, n), lambda l, i: (i, 0)),
            pl.BlockSpec((n, fin), lambda l, i: (0, 0)),
            pl.BlockSpec((fin, f), lambda l, i: (0, 0)),
            pl.BlockSpec((f, f), lambda l, i: (0, 0)),
            pl.BlockSpec((1, f), lambda l, i: (0, 0)),
            pl.BlockSpec((1, f), lambda l, i: (0, 0)),
        ],
        out_specs=pl.BlockSpec((_BR2, f), lambda l, i: (i, 0)),
        out_shape=jax.ShapeDtypeStruct((n, f), jnp.float32),
        scratch_shapes=[
            pltpu.VMEM((n, f), jnp.bfloat16),
            pltpu.VMEM((n, f), jnp.float32),
        ],
    )(adjb, h1, w2, w3, b2, b3)


def kernel(x, adj, W1, b1, W2, b2, W3, b3):
    h1, adjb = _layer1(adj, x, W1, b1.reshape(1, -1))
    out = _layer23(adjb, h1, W2, W3, b2.reshape(1, -1), b3.reshape(1, -1))
    return out
